# Initial kernel scaffold; baseline (speedup 1.0000x reference)
import jax
import jax.numpy as jnp
from jax.experimental import pallas as pl


def kernel(user_tensor, item_tensor, user_emb, item_emb, edge_w, src, dst):
    raise NotImplementedError("write your pallas kernel here")


# speedup vs baseline: 4.2394x; 4.2394x over previous
import functools

import jax
import jax.numpy as jnp
from jax import lax
from jax.experimental import pallas as pl
from jax.experimental.pallas import tpu as pltpu
from jax.experimental.pallas import tpu_sc as plsc

NU = 25000
NN = 2 * NU
D = 64
E_HALF = 400000
CH = 128
GC = 8
NGROUP = 25
NCHUNK = NGROUP * GC
EPT = NCHUNK * CH
ROWS_A, ROWS_B = 1568, 1560

_MESH = plsc.VectorSubcoreMesh(core_axis_name="c", subcore_axis_name="s")


def _layer_body(emb, srcp, dstp, wp, out, src_v, dst_v, w_v, rows_v, acc_sh, sem):
    c = lax.axis_index("c")
    s = lax.axis_index("s")
    blk = c * 16 + s
    base = c * NU

    zero16 = jnp.zeros((16,), jnp.float32)

    def _zbuf(r, carry):
        for j in range(4):
            rows_v[r, pl.ds(j * 16, 16)] = zero16
        return carry

    lax.fori_loop(0, CH, _zbuf, 0)

    row0 = pl.multiple_of(
        jnp.where(s < 5, s * ROWS_A, 5 * ROWS_A + (s - 5) * ROWS_B), 8)

    def _ranged_copy(copy_one):
        def _chunks(k, carry):
            copy_one(k * CH, CH)
            return carry

        lax.fori_loop(0, 12, _chunks, 0)

        @pl.when(s < 5)
        def _tail_a():
            copy_one(1536, ROWS_A - 1536)

        @pl.when(s >= 5)
        def _tail_b():
            copy_one(1536, ROWS_B - 1536)

    _ranged_copy(lambda off, n: pltpu.sync_copy(
        rows_v.at[pl.ds(0, n)], acc_sh.at[pl.ds(row0 + off, n)]))

    plsc.subcore_barrier()

    def _group(gr, carry):
        gsl = pl.ds(gr * GC, GC)
        pltpu.sync_copy(srcp.at[blk, gsl], src_v)
        pltpu.sync_copy(dstp.at[blk, gsl], dst_v)
        pltpu.sync_copy(wp.at[blk, gsl], w_v)

        def _localize(k, c3):
            for j in range(8):
                sl = pl.ds(j * 16, 16)
                dst_v[k, sl] = dst_v[k, sl] - base
            return c3

        lax.fori_loop(0, GC, _localize, 0)

        def _edge(k, c2):
            pltpu.async_copy(emb.at[src_v.at[k]], rows_v, sem).wait()

            def _mul(g, c4):
                wvec = w_v[k, pl.ds(g * 16, 16)]
                for i in range(16):
                    r = g * 16 + i
                    wv = wvec[i]
                    for j in range(4):
                        sl = pl.ds(j * 16, 16)
                        rows_v[r, sl] = rows_v[r, sl] * wv
                return c4

            lax.fori_loop(0, CH // 16, _mul, 0)
            pltpu.sync_copy(rows_v, acc_sh.at[dst_v.at[k]], add=True)
            return c2

        lax.fori_loop(0, GC, _edge, 0)
        return carry

    lax.fori_loop(0, NGROUP, _group, 0)
    plsc.subcore_barrier()

    gbase = base + row0
    _ranged_copy(lambda off, n: pltpu.sync_copy(
        acc_sh.at[pl.ds(row0 + off, n)], out.at[pl.ds(gbase + off, n)]))


_layer = functools.partial(
    pl.kernel,
    mesh=_MESH,
    compiler_params=pltpu.CompilerParams(use_tc_tiling_on_sc=False),
    out_type=jax.ShapeDtypeStruct((NN, D), jnp.float32),
    scratch_types=[
        pltpu.VMEM((GC, CH), jnp.int32),
        pltpu.VMEM((GC, CH), jnp.int32),
        pltpu.VMEM((GC, CH), jnp.float32),
        pltpu.VMEM((CH, D), jnp.float32),
        pltpu.VMEM_SHARED((NU, D), jnp.float32),
        pltpu.SemaphoreType.DMA,
    ],
)(_layer_body)


def _avg_body(e0, e1, e2, e3, idxp, out, idx_v, acc_v, rows_v, sem):
    c = lax.axis_index("c")
    s = lax.axis_index("s")
    w = c * 16 + s
    pltpu.sync_copy(idxp.at[w], idx_v)
    for j in range(2):
        pltpu.async_copy(e0.at[idx_v.at[j]], acc_v, sem).wait()
        for e in (e1, e2, e3):
            pltpu.async_copy(e.at[idx_v.at[j]], rows_v, sem).wait()

            def _add(r, carry):
                for g in range(4):
                    sl = pl.ds(g * 16, 16)
                    acc_v[r, sl] = acc_v[r, sl] + rows_v[r, sl]
                return carry

            lax.fori_loop(0, 80, _add, 0)

        def _scale(r, carry):
            for g in range(4):
                sl = pl.ds(g * 16, 16)
                acc_v[r, sl] = acc_v[r, sl] * 0.25
            return carry

        lax.fori_loop(0, 80, _scale, 0)
        pltpu.sync_copy(acc_v, out.at[pl.ds(w * 160 + j * 80, 80)])


_avg = functools.partial(
    pl.kernel,
    mesh=_MESH,
    compiler_params=pltpu.CompilerParams(use_tc_tiling_on_sc=False),
    out_type=jax.ShapeDtypeStruct((5120, D), jnp.float32),
    scratch_types=[
        pltpu.VMEM((2, 80), jnp.int32),
        pltpu.VMEM((80, D), jnp.float32),
        pltpu.VMEM((80, D), jnp.float32),
        pltpu.SemaphoreType.DMA,
    ],
)(_avg_body)


def _mm_body(u_ref, it_ref, o_ref):
    x = lax.dot_general(u_ref[...], it_ref[...], (((1,), (1,)), ((), ())),
                        preferred_element_type=jnp.float32)
    o_ref[...] = 1.0 / (1.0 + jnp.exp(-x))


def kernel(user_tensor, item_tensor, user_emb, item_emb, edge_w, src, dst):
    all_emb = jnp.concatenate([user_emb, item_emb], axis=0)

    src_r = jnp.concatenate([src[E_HALF:], src[:E_HALF]]).astype(jnp.int32)
    dst_r = jnp.concatenate([dst[E_HALF:], dst[:E_HALF]]).astype(jnp.int32)
    w_r = jnp.concatenate([edge_w[E_HALF:], edge_w[:E_HALF]])

    pad = EPT - NU
    src_p = jnp.pad(src_r.reshape(32, NU), ((0, 0), (0, pad))).reshape(32, NCHUNK, CH)
    w_p = jnp.pad(w_r.reshape(32, NU), ((0, 0), (0, pad))).reshape(32, NCHUNK, CH)
    dpad = jnp.where(jnp.arange(32) < 16, 0, NU).astype(jnp.int32)
    dst_p = jnp.concatenate(
        [dst_r.reshape(32, NU), jnp.broadcast_to(dpad[:, None], (32, pad))],
        axis=1).reshape(32, NCHUNK, CH)

    e0 = all_emb
    e1 = _layer(e0, src_p, dst_p, w_p)
    e2 = _layer(e1, src_p, dst_p, w_p)
    e3 = _layer(e2, src_p, dst_p, w_p)

    idx = jnp.concatenate([user_tensor.astype(jnp.int32),
                           item_tensor.astype(jnp.int32) + NU])
    vecs = _avg(e0, e1, e2, e3, idx.reshape(32, 2, 80))

    user_vec = vecs[:1024]
    item_vec = vecs[1024:]
    return pl.pallas_call(
        _mm_body,
        out_shape=jax.ShapeDtypeStruct((1024, 4096), jnp.float32),
    )(user_vec, item_vec)
